# Initial kernel scaffold; baseline (speedup 1.0000x reference)
import jax
import jax.numpy as jnp
from jax.experimental import pallas as pl


def kernel(x, prob):
    raise NotImplementedError("write your pallas kernel here")


# speedup vs baseline: 1.0341x; 1.0341x over previous
import functools

import jax
import jax.numpy as jnp
from jax import lax
from jax.experimental import pallas as pl
from jax.experimental.pallas import tpu as pltpu
from jax.experimental.pallas import tpu_sc as plsc

_B = 1024
_L = 50
_V = 1000
_D = 1000
_N = _B * _L

_NC = 2
_NS = 16
_NW = _NC * _NS
_ROWS_PER_W = _N // _NW
_CHUNK = 40
_NCHUNK = _ROWS_PER_W // _CHUNK


def _make_gather():
    mesh = plsc.VectorSubcoreMesh(core_axis_name="c", subcore_axis_name="s")

    @functools.partial(
        pl.kernel,
        mesh=mesh,
        out_type=jax.ShapeDtypeStruct((_N, _D), jnp.float32),
        scratch_types=[
            pltpu.VMEM((_ROWS_PER_W,), jnp.int32),
            pltpu.VMEM((_CHUNK, _D), jnp.float32),
            pltpu.VMEM((_CHUNK, _D), jnp.float32),
            pltpu.SemaphoreType.DMA,
            pltpu.SemaphoreType.DMA,
        ],
        compiler_params=pltpu.CompilerParams(use_tc_tiling_on_sc=False),
    )
    def gather_kernel(idx_hbm, tab_hbm, out_hbm, idx_v, buf0, buf1, sem0, sem1):
        wid = lax.axis_index("s") * _NC + lax.axis_index("c")
        base = wid * _ROWS_PER_W

        pltpu.sync_copy(idx_hbm.at[pl.ds(base, _ROWS_PER_W)], idx_v)

        bufs = (buf0, buf1)
        sems = (sem0, sem1)

        def start_gather(c, b):
            pltpu.async_copy(
                tab_hbm.at[idx_v.at[pl.ds(c * _CHUNK, _CHUNK)]], bufs[b], sems[b]
            )

        def finish(c, b):
            pltpu.make_async_copy(
                tab_hbm.at[idx_v.at[pl.ds(c * _CHUNK, _CHUNK)]], bufs[b], sems[b]
            ).wait()
            pltpu.sync_copy(
                bufs[b], out_hbm.at[pl.ds(base + c * _CHUNK, _CHUNK)]
            )

        start_gather(0, 0)

        def body(g, carry):
            c0 = 2 * g
            start_gather(c0 + 1, 1)
            finish(c0, 0)
            @pl.when(c0 + 2 < _NCHUNK)
            def _():
                start_gather(c0 + 2, 0)
            finish(c0 + 1, 1)
            return carry

        lax.fori_loop(0, _NCHUNK // 2, body, 0)

    return gather_kernel


_gather = _make_gather()


def kernel(x, prob):
    flat_idx = x.reshape(-1).astype(jnp.int32)
    out = _gather(flat_idx, prob)
    return out.reshape(x.shape[0], x.shape[1], _D)


# speedup vs baseline: 1.1404x; 1.1028x over previous
import functools

import jax
import jax.numpy as jnp
from jax import lax
from jax.experimental import pallas as pl
from jax.experimental.pallas import tpu as pltpu
from jax.experimental.pallas import tpu_sc as plsc

_B = 1024
_L = 50
_V = 1000
_D = 1000
_N = _B * _L

_NC = 2
_NS = 16
_NW = _NC * _NS
_ROWS_PER_W = _N // _NW
_CHUNK = 32
_NCHUNK = _ROWS_PER_W // _CHUNK

_STAGERS = 8
_STAGE_ROWS = _V // _STAGERS


def _make_gather():
    mesh = plsc.VectorSubcoreMesh(core_axis_name="c", subcore_axis_name="s")

    @functools.partial(
        pl.kernel,
        mesh=mesh,
        out_type=jax.ShapeDtypeStruct((_N, _D), jnp.float32),
        scratch_types=[
            pltpu.VMEM_SHARED((_V, _D), jnp.float32),
            pltpu.VMEM((_ROWS_PER_W,), jnp.int32),
            pltpu.VMEM((_CHUNK, _D), jnp.float32),
            pltpu.VMEM((_CHUNK, _D), jnp.float32),
            pltpu.SemaphoreType.DMA,
            pltpu.SemaphoreType.DMA,
        ],
        compiler_params=pltpu.CompilerParams(use_tc_tiling_on_sc=False),
    )
    def gather_kernel(idx_hbm, tab_hbm, out_hbm, shared, idx_v, buf0, buf1,
                      sem0, sem1):
        sid = lax.axis_index("s")
        wid = sid * _NC + lax.axis_index("c")
        base = wid * _ROWS_PER_W

        pltpu.sync_copy(idx_hbm.at[pl.ds(base, _ROWS_PER_W)], idx_v)

        @pl.when(sid < _STAGERS)
        def _():
            pltpu.sync_copy(
                tab_hbm.at[pl.ds(sid * _STAGE_ROWS, _STAGE_ROWS)],
                shared.at[pl.ds(sid * _STAGE_ROWS, _STAGE_ROWS)],
            )

        plsc.subcore_barrier()

        bufs = (buf0, buf1)
        sems = (sem0, sem1)

        def start_gather(c, b):
            pltpu.async_copy(
                shared.at[idx_v.at[pl.ds(c * _CHUNK, _CHUNK)]], bufs[b], sems[b]
            )

        def finish(c, b):
            pltpu.make_async_copy(
                shared.at[idx_v.at[pl.ds(c * _CHUNK, _CHUNK)]], bufs[b], sems[b]
            ).wait()
            pltpu.sync_copy(
                bufs[b], out_hbm.at[pl.ds(base + c * _CHUNK, _CHUNK)]
            )

        start_gather(0, 0)

        def body(g, carry):
            c0 = 2 * g
            start_gather(c0 + 1, 1)
            finish(c0, 0)
            @pl.when(c0 + 2 < _NCHUNK)
            def _():
                start_gather(c0 + 2, 0)
            finish(c0 + 1, 1)
            return carry

        lax.fori_loop(0, _NCHUNK // 2, body, 0)

    return gather_kernel


_gather = _make_gather()


def kernel(x, prob):
    flat_idx = x.reshape(-1).astype(jnp.int32)
    out = _gather(flat_idx, prob)
    return out.reshape(x.shape[0], x.shape[1], _D)


# speedup vs baseline: 1.1406x; 1.0001x over previous
import functools

import jax
import jax.numpy as jnp
from jax import lax
from jax.experimental import pallas as pl
from jax.experimental.pallas import tpu as pltpu
from jax.experimental.pallas import tpu_sc as plsc

_B = 1024
_L = 50
_V = 1000
_D = 1000
_N = _B * _L

_NC = 2
_NS = 16
_NW = _NC * _NS
_ROWS_PER_W = _N // _NW
_CHUNK = 16
_NBUF = 4
_NCHUNK = _ROWS_PER_W // _CHUNK
_NROUND = _NCHUNK // _NBUF

_STAGERS = 8
_STAGE_ROWS = _V // _STAGERS


def _make_gather():
    mesh = plsc.VectorSubcoreMesh(core_axis_name="c", subcore_axis_name="s")

    @functools.partial(
        pl.kernel,
        mesh=mesh,
        out_type=jax.ShapeDtypeStruct((_N, _D), jnp.float32),
        scratch_types=[
            pltpu.VMEM_SHARED((_V, _D), jnp.float32),
            pltpu.VMEM((_ROWS_PER_W,), jnp.int32),
        ]
        + [pltpu.VMEM((_CHUNK, _D), jnp.float32) for _ in range(_NBUF)]
        + [pltpu.SemaphoreType.DMA for _ in range(2 * _NBUF)],
        compiler_params=pltpu.CompilerParams(use_tc_tiling_on_sc=False),
    )
    def gather_kernel(idx_hbm, tab_hbm, out_hbm, shared, idx_v, *rest):
        bufs = rest[:_NBUF]
        gsems = rest[_NBUF:2 * _NBUF]
        wsems = rest[2 * _NBUF:]

        sid = lax.axis_index("s")
        wid = sid * _NC + lax.axis_index("c")
        base = wid * _ROWS_PER_W

        pltpu.sync_copy(idx_hbm.at[pl.ds(base, _ROWS_PER_W)], idx_v)

        @pl.when(sid < _STAGERS)
        def _():
            pltpu.sync_copy(
                tab_hbm.at[pl.ds(sid * _STAGE_ROWS, _STAGE_ROWS)],
                shared.at[pl.ds(sid * _STAGE_ROWS, _STAGE_ROWS)],
            )

        plsc.subcore_barrier()

        def start_gather(c, b):
            pltpu.async_copy(
                shared.at[idx_v.at[pl.ds(c * _CHUNK, _CHUNK)]], bufs[b], gsems[b]
            )

        def wait_gather(c, b):
            pltpu.make_async_copy(
                shared.at[idx_v.at[pl.ds(c * _CHUNK, _CHUNK)]], bufs[b], gsems[b]
            ).wait()

        def start_write(c, b):
            pltpu.async_copy(
                bufs[b], out_hbm.at[pl.ds(base + c * _CHUNK, _CHUNK)], wsems[b]
            )

        def wait_write(c, b):
            pltpu.make_async_copy(
                bufs[b], out_hbm.at[pl.ds(base + c * _CHUNK, _CHUNK)], wsems[b]
            ).wait()

        for b in range(_NBUF):
            start_gather(b, b)

        def body(r, carry):
            cb = r * _NBUF
            for b in range(_NBUF):
                wait_gather(cb + b, b)
                start_write(cb + b, b)
            @pl.when(r + 1 < _NROUND)
            def _():
                for b in range(_NBUF):
                    wait_write(cb + b, b)
                    start_gather(cb + _NBUF + b, b)
            return carry

        lax.fori_loop(0, _NROUND, body, 0)

        for b in range(_NBUF):
            wait_write(_NCHUNK - _NBUF + b, b)

    return gather_kernel


_gather = _make_gather()


def kernel(x, prob):
    flat_idx = x.reshape(-1).astype(jnp.int32)
    out = _gather(flat_idx, prob)
    return out.reshape(x.shape[0], x.shape[1], _D)
